# baseline (speedup 1.0000x reference)
import functools

import jax
import jax.numpy as jnp
from jax import lax
from jax.experimental import pallas as pl
from jax.experimental.pallas import tpu as pltpu

_LANE = 128


def _pick_tile(hw_pad, max_tile):
    t = min(hw_pad, max_tile) // _LANE * _LANE
    while t > _LANE and hw_pad % t != 0:
        t -= _LANE
    return max(t, _LANE)


def _loss_body(x_ref, w_ref, b_ref, o_ref, *, inv_n, tile, hw_valid, masked):
    j = pl.program_id(1)

    @pl.when(j == 0)
    def _():
        o_ref[...] = jnp.zeros_like(o_ref)

    xb = x_ref[0].astype(jnp.bfloat16)
    feat = jnp.dot(w_ref[...], xb,
                   preferred_element_type=jnp.float32)
    feat = feat + b_ref[...]
    sq = feat * feat
    if masked:
        col = j * tile + lax.broadcasted_iota(jnp.int32, (1, tile), 1)
        sq = jnp.where(col < hw_valid, sq, 0.0)
    o_ref[...] += jnp.sum(sq) * inv_n


def kernel(img, weight, bias):
    N, C, H, W = img.shape
    Cout = weight.shape[0]
    hw = H * W
    hw_pad = -(-hw // _LANE) * _LANE
    x3 = img.reshape(N, C, hw)
    if hw_pad != hw:
        x3 = jnp.pad(x3, ((0, 0), (0, 0), (0, hw_pad - hw)))
    T = _pick_tile(hw_pad, 4096)
    inv_n = 1.0 / float(N * Cout * hw)
    w_bf = weight.astype(jnp.bfloat16)

    partials = pl.pallas_call(
        functools.partial(_loss_body, inv_n=inv_n, tile=T, hw_valid=hw,
                          masked=hw_pad != hw),
        out_shape=jax.ShapeDtypeStruct((N, 1, 1), jnp.float32),
        grid=(N, hw_pad // T),
        in_specs=[
            pl.BlockSpec((1, C, T), lambda n, j: (n, 0, j)),
            pl.BlockSpec((Cout, C), lambda n, j: (0, 0)),
            pl.BlockSpec((Cout, 1), lambda n, j: (0, 0)),
        ],
        out_specs=pl.BlockSpec((1, 1, 1), lambda n, j: (n, 0, 0)),
        compiler_params=pltpu.CompilerParams(
            dimension_semantics=("parallel", "arbitrary"),
            vmem_limit_bytes=48 * 1024 * 1024),
    )(x3, w_bf, bias)
    return jnp.sum(partials)


# speedup vs baseline: 1.0487x; 1.0487x over previous
import functools

import jax
import jax.numpy as jnp
from jax import lax
from jax.experimental import pallas as pl
from jax.experimental.pallas import tpu as pltpu

_LANE = 128


def _pick_tile(hw_pad, max_tile):
    t = min(hw_pad, max_tile) // _LANE * _LANE
    while t > _LANE and hw_pad % t != 0:
        t -= _LANE
    return max(t, _LANE)


def _loss_body(x_ref, w_ref, b_ref, o_ref, *, inv_n, tile, hw_valid, masked):
    j = pl.program_id(1)

    @pl.when(j == 0)
    def _():
        o_ref[...] = jnp.zeros_like(o_ref)

    xb = x_ref[0].astype(jnp.bfloat16)
    feat = jnp.dot(w_ref[...], xb,
                   preferred_element_type=jnp.float32)
    feat = feat + b_ref[...]
    sq = feat * feat
    if masked:
        col = j * tile + lax.broadcasted_iota(jnp.int32, (1, tile), 1)
        sq = jnp.where(col < hw_valid, sq, 0.0)
    o_ref[...] += jnp.sum(sq) * inv_n


def kernel(img, weight, bias):
    N, C, H, W = img.shape
    Cout = weight.shape[0]
    hw = H * W
    hw_pad = -(-hw // _LANE) * _LANE
    x3 = img.reshape(N, C, hw)
    if hw_pad != hw:
        x3 = jnp.pad(x3, ((0, 0), (0, 0), (0, hw_pad - hw)))
    T = _pick_tile(hw_pad, 16384)
    inv_n = 1.0 / float(N * Cout * hw)
    w_bf = weight.astype(jnp.bfloat16)

    partials = pl.pallas_call(
        functools.partial(_loss_body, inv_n=inv_n, tile=T, hw_valid=hw,
                          masked=hw_pad != hw),
        out_shape=jax.ShapeDtypeStruct((N, 1, 1), jnp.float32),
        grid=(N, hw_pad // T),
        in_specs=[
            pl.BlockSpec((1, C, T), lambda n, j: (n, 0, j)),
            pl.BlockSpec((Cout, C), lambda n, j: (0, 0)),
            pl.BlockSpec((Cout, 1), lambda n, j: (0, 0)),
        ],
        out_specs=pl.BlockSpec((1, 1, 1), lambda n, j: (n, 0, 0)),
        compiler_params=pltpu.CompilerParams(
            dimension_semantics=("parallel", "arbitrary"),
            vmem_limit_bytes=48 * 1024 * 1024),
    )(x3, w_bf, bias)
    return jnp.sum(partials)


# speedup vs baseline: 1.0495x; 1.0007x over previous
import functools

import jax
import jax.numpy as jnp
from jax import lax
from jax.experimental import pallas as pl
from jax.experimental.pallas import tpu as pltpu

_LANE = 128


def _pick_tile(hw_pad, max_tile):
    t = min(hw_pad, max_tile) // _LANE * _LANE
    while t > _LANE and hw_pad % t != 0:
        t -= _LANE
    return max(t, _LANE)


def _loss_body(x_ref, w_ref, b_ref, o_ref, *, inv_n, tile, hw_valid, masked):
    j = pl.program_id(1)

    @pl.when(j == 0)
    def _():
        o_ref[...] = jnp.zeros_like(o_ref)

    feat = jnp.dot(w_ref[...], x_ref[0],
                   preferred_element_type=jnp.float32)
    feat = feat + b_ref[...]
    sq = feat * feat
    if masked:
        col = j * tile + lax.broadcasted_iota(jnp.int32, (1, tile), 1)
        sq = jnp.where(col < hw_valid, sq, 0.0)
    o_ref[...] += jnp.sum(sq) * inv_n


def kernel(img, weight, bias):
    N, C, H, W = img.shape
    Cout = weight.shape[0]
    hw = H * W
    hw_pad = -(-hw // _LANE) * _LANE
    x3 = img.reshape(N, C, hw)
    if hw_pad != hw:
        x3 = jnp.pad(x3, ((0, 0), (0, 0), (0, hw_pad - hw)))
    T = _pick_tile(hw_pad, 16384)
    inv_n = 1.0 / float(N * Cout * hw)
    w_bf = weight

    partials = pl.pallas_call(
        functools.partial(_loss_body, inv_n=inv_n, tile=T, hw_valid=hw,
                          masked=hw_pad != hw),
        out_shape=jax.ShapeDtypeStruct((N, 1, 1), jnp.float32),
        grid=(N, hw_pad // T),
        in_specs=[
            pl.BlockSpec((1, C, T), lambda n, j: (n, 0, j)),
            pl.BlockSpec((Cout, C), lambda n, j: (0, 0)),
            pl.BlockSpec((Cout, 1), lambda n, j: (0, 0)),
        ],
        out_specs=pl.BlockSpec((1, 1, 1), lambda n, j: (n, 0, 0)),
        compiler_params=pltpu.CompilerParams(
            dimension_semantics=("parallel", "arbitrary"),
            vmem_limit_bytes=48 * 1024 * 1024),
    )(x3, w_bf, bias)
    return jnp.sum(partials)


# speedup vs baseline: 1.1310x; 1.0778x over previous
import functools

import jax
import jax.numpy as jnp
from jax import lax
from jax.experimental import pallas as pl
from jax.experimental.pallas import tpu as pltpu

_LANE = 128


def _pick_tile(hw_pad, max_tile):
    t = min(hw_pad, max_tile) // _LANE * _LANE
    while t > _LANE and hw_pad % t != 0:
        t -= _LANE
    return max(t, _LANE)


def _gram_body(x_ref, w_ref, b_ref, o_ref, g_ref, s_ref, *, nsteps,
               batch_hw, inv_n):
    step = pl.program_id(0) * pl.num_programs(1) + pl.program_id(1)

    @pl.when(step == 0)
    def _():
        g_ref[...] = jnp.zeros_like(g_ref)
        s_ref[...] = jnp.zeros_like(s_ref)

    x = x_ref[0]
    g_ref[...] += lax.dot_general(x, x, (((1,), (1,)), ((), ())),
                                  preferred_element_type=jnp.float32)
    s_ref[...] += jnp.sum(x, axis=1, keepdims=True)

    @pl.when(step == nsteps - 1)
    def _():
        w = w_ref[...]
        b = b_ref[...]
        wg = jnp.dot(w, g_ref[...], preferred_element_type=jnp.float32)
        quad = jnp.sum(wg * w)
        ws = jnp.dot(w, s_ref[...], preferred_element_type=jnp.float32)
        cross = 2.0 * jnp.sum(ws * b)
        const = batch_hw * jnp.sum(b * b)
        o_ref[...] = jnp.reshape((quad + cross + const) * inv_n, (1, 1))


def kernel(img, weight, bias):
    N, C, H, W = img.shape
    Cout = weight.shape[0]
    hw = H * W
    hw_pad = -(-hw // _LANE) * _LANE
    x3 = img.reshape(N, C, hw)
    if hw_pad != hw:
        x3 = jnp.pad(x3, ((0, 0), (0, 0), (0, hw_pad - hw)))
    T = _pick_tile(hw_pad, 16384)
    steps_j = hw_pad // T

    out = pl.pallas_call(
        functools.partial(_gram_body, nsteps=N * steps_j,
                          batch_hw=float(N * hw), inv_n=1.0 / float(N * Cout * hw)),
        out_shape=jax.ShapeDtypeStruct((1, 1), jnp.float32),
        grid=(N, steps_j),
        in_specs=[
            pl.BlockSpec((1, C, T), lambda n, j: (n, 0, j)),
            pl.BlockSpec((Cout, C), lambda n, j: (0, 0)),
            pl.BlockSpec((Cout, 1), lambda n, j: (0, 0)),
        ],
        out_specs=pl.BlockSpec((1, 1), lambda n, j: (0, 0)),
        scratch_shapes=[
            pltpu.VMEM((C, C), jnp.float32),
            pltpu.VMEM((C, 1), jnp.float32),
        ],
        compiler_params=pltpu.CompilerParams(
            dimension_semantics=("arbitrary", "arbitrary"),
            vmem_limit_bytes=48 * 1024 * 1024),
    )(x3, weight, bias)
    return out[0, 0]


# speedup vs baseline: 1.1311x; 1.0000x over previous
import functools

import jax
import jax.numpy as jnp
from jax import lax
from jax.experimental import pallas as pl
from jax.experimental.pallas import tpu as pltpu

_LANE = 128


def _pick_tile(hw_pad, max_tile):
    t = min(hw_pad, max_tile) // _LANE * _LANE
    while t > _LANE and hw_pad % t != 0:
        t -= _LANE
    return max(t, _LANE)


def _gram_body(x_ref, w_ref, b_ref, o_ref, g_ref, s_ref, *, nsteps,
               batch_hw, inv_n):
    step = pl.program_id(0) * pl.num_programs(1) + pl.program_id(1)

    @pl.when(step == 0)
    def _():
        g_ref[...] = jnp.zeros_like(g_ref)
        s_ref[...] = jnp.zeros_like(s_ref)

    x = x_ref[0]
    g_ref[...] += lax.dot_general(x, x, (((1,), (1,)), ((), ())),
                                  preferred_element_type=jnp.float32)
    s_ref[...] += jnp.sum(x, axis=1, keepdims=True)

    @pl.when(step == nsteps - 1)
    def _():
        w = w_ref[...]
        b = b_ref[...]
        wg = jnp.dot(w, g_ref[...], preferred_element_type=jnp.float32)
        quad = jnp.sum(wg * w)
        ws = jnp.dot(w, s_ref[...], preferred_element_type=jnp.float32)
        cross = 2.0 * jnp.sum(ws * b)
        const = batch_hw * jnp.sum(b * b)
        o_ref[...] = jnp.reshape((quad + cross + const) * inv_n, (1, 1))


def kernel(img, weight, bias):
    N, C, H, W = img.shape
    Cout = weight.shape[0]
    hw = H * W
    hw_pad = -(-hw // _LANE) * _LANE
    x3 = img.reshape(N, C, hw)
    if hw_pad != hw:
        x3 = jnp.pad(x3, ((0, 0), (0, 0), (0, hw_pad - hw)))
    T = _pick_tile(hw_pad, 8192)
    steps_j = hw_pad // T

    out = pl.pallas_call(
        functools.partial(_gram_body, nsteps=N * steps_j,
                          batch_hw=float(N * hw), inv_n=1.0 / float(N * Cout * hw)),
        out_shape=jax.ShapeDtypeStruct((1, 1), jnp.float32),
        grid=(N, steps_j),
        in_specs=[
            pl.BlockSpec((1, C, T), lambda n, j: (n, 0, j)),
            pl.BlockSpec((Cout, C), lambda n, j: (0, 0)),
            pl.BlockSpec((Cout, 1), lambda n, j: (0, 0)),
        ],
        out_specs=pl.BlockSpec((1, 1), lambda n, j: (0, 0)),
        scratch_shapes=[
            pltpu.VMEM((C, C), jnp.float32),
            pltpu.VMEM((C, 1), jnp.float32),
        ],
        compiler_params=pltpu.CompilerParams(
            dimension_semantics=("arbitrary", "arbitrary"),
            vmem_limit_bytes=48 * 1024 * 1024),
    )(x3, weight, bias)
    return out[0, 0]


# speedup vs baseline: 1.1346x; 1.0031x over previous
import functools

import jax
import jax.numpy as jnp
from jax import lax
from jax.experimental import pallas as pl
from jax.experimental.pallas import tpu as pltpu

_LANE = 128


def _pick_tile(hw_pad, max_tile):
    t = min(hw_pad, max_tile) // _LANE * _LANE
    while t > _LANE and hw_pad % t != 0:
        t -= _LANE
    return max(t, _LANE)


def _gram_body(x_ref, w_ref, b_ref, o_ref, g_ref, s_ref, *, nsteps,
               batch_hw, inv_n):
    step = pl.program_id(0) * pl.num_programs(1) + pl.program_id(1)

    @pl.when(step == 0)
    def _():
        g_ref[...] = jnp.zeros_like(g_ref)
        s_ref[...] = jnp.zeros_like(s_ref)

    x = x_ref[0]
    xb = x.astype(jnp.bfloat16)
    g_ref[...] += lax.dot_general(xb, xb, (((1,), (1,)), ((), ())),
                                  preferred_element_type=jnp.float32)
    s_ref[...] += jnp.sum(x, axis=1, keepdims=True)

    @pl.when(step == nsteps - 1)
    def _():
        w = w_ref[...]
        b = b_ref[...]
        wg = jnp.dot(w, g_ref[...], preferred_element_type=jnp.float32)
        quad = jnp.sum(wg * w)
        ws = jnp.dot(w, s_ref[...], preferred_element_type=jnp.float32)
        cross = 2.0 * jnp.sum(ws * b)
        const = batch_hw * jnp.sum(b * b)
        o_ref[...] = jnp.reshape((quad + cross + const) * inv_n, (1, 1))


def kernel(img, weight, bias):
    N, C, H, W = img.shape
    Cout = weight.shape[0]
    hw = H * W
    hw_pad = -(-hw // _LANE) * _LANE
    x3 = img.reshape(N, C, hw)
    if hw_pad != hw:
        x3 = jnp.pad(x3, ((0, 0), (0, 0), (0, hw_pad - hw)))
    T = _pick_tile(hw_pad, 16384)
    steps_j = hw_pad // T

    out = pl.pallas_call(
        functools.partial(_gram_body, nsteps=N * steps_j,
                          batch_hw=float(N * hw), inv_n=1.0 / float(N * Cout * hw)),
        out_shape=jax.ShapeDtypeStruct((1, 1), jnp.float32),
        grid=(N, steps_j),
        in_specs=[
            pl.BlockSpec((1, C, T), lambda n, j: (n, 0, j)),
            pl.BlockSpec((Cout, C), lambda n, j: (0, 0)),
            pl.BlockSpec((Cout, 1), lambda n, j: (0, 0)),
        ],
        out_specs=pl.BlockSpec((1, 1), lambda n, j: (0, 0)),
        scratch_shapes=[
            pltpu.VMEM((C, C), jnp.float32),
            pltpu.VMEM((C, 1), jnp.float32),
        ],
        compiler_params=pltpu.CompilerParams(
            dimension_semantics=("arbitrary", "arbitrary"),
            vmem_limit_bytes=48 * 1024 * 1024),
    )(x3, weight, bias)
    return out[0, 0]


# speedup vs baseline: 1.1352x; 1.0006x over previous
import functools

import jax
import jax.numpy as jnp
from jax import lax
from jax.experimental import pallas as pl
from jax.experimental.pallas import tpu as pltpu

_LANE = 128


def _pick_tile(hw_pad, max_tile):
    t = min(hw_pad, max_tile) // _LANE * _LANE
    while t > _LANE and hw_pad % t != 0:
        t -= _LANE
    return max(t, _LANE)


def _gram_body(x_ref, w_ref, b_ref, o_ref, g_ref, s_ref, *, nsteps,
               batch_hw, inv_n):
    step = pl.program_id(0) * pl.num_programs(1) + pl.program_id(1)

    @pl.when(step == 0)
    def _():
        g_ref[...] = jnp.zeros_like(g_ref)
        s_ref[...] = jnp.zeros_like(s_ref)

    x = x_ref[0]
    xb = x.astype(jnp.bfloat16)
    g_ref[...] += lax.dot_general(xb, xb, (((1,), (1,)), ((), ())),
                                  preferred_element_type=jnp.float32)
    s_ref[...] += jnp.sum(xb, axis=1, keepdims=True, dtype=jnp.float32)

    @pl.when(step == nsteps - 1)
    def _():
        w = w_ref[...]
        b = b_ref[...]
        wg = jnp.dot(w, g_ref[...], preferred_element_type=jnp.float32)
        quad = jnp.sum(wg * w)
        ws = jnp.dot(w, s_ref[...], preferred_element_type=jnp.float32)
        cross = 2.0 * jnp.sum(ws * b)
        const = batch_hw * jnp.sum(b * b)
        o_ref[...] = jnp.reshape((quad + cross + const) * inv_n, (1, 1))


def kernel(img, weight, bias):
    N, C, H, W = img.shape
    Cout = weight.shape[0]
    hw = H * W
    hw_pad = -(-hw // _LANE) * _LANE
    x3 = img.reshape(N, C, hw)
    if hw_pad != hw:
        x3 = jnp.pad(x3, ((0, 0), (0, 0), (0, hw_pad - hw)))
    T = _pick_tile(hw_pad, 16384)
    steps_j = hw_pad // T

    out = pl.pallas_call(
        functools.partial(_gram_body, nsteps=N * steps_j,
                          batch_hw=float(N * hw), inv_n=1.0 / float(N * Cout * hw)),
        out_shape=jax.ShapeDtypeStruct((1, 1), jnp.float32),
        grid=(N, steps_j),
        in_specs=[
            pl.BlockSpec((1, C, T), lambda n, j: (n, 0, j)),
            pl.BlockSpec((Cout, C), lambda n, j: (0, 0)),
            pl.BlockSpec((Cout, 1), lambda n, j: (0, 0)),
        ],
        out_specs=pl.BlockSpec((1, 1), lambda n, j: (0, 0)),
        scratch_shapes=[
            pltpu.VMEM((C, C), jnp.float32),
            pltpu.VMEM((C, 1), jnp.float32),
        ],
        compiler_params=pltpu.CompilerParams(
            dimension_semantics=("arbitrary", "arbitrary"),
            vmem_limit_bytes=48 * 1024 * 1024),
    )(x3, weight, bias)
    return out[0, 0]


# speedup vs baseline: 1.1416x; 1.0056x over previous
import functools

import jax
import jax.numpy as jnp
from jax import lax
from jax.experimental import pallas as pl
from jax.experimental.pallas import tpu as pltpu

_LANE = 128


def _pick_tile(hw_pad, max_tile):
    t = min(hw_pad, max_tile) // _LANE * _LANE
    while t > _LANE and hw_pad % t != 0:
        t -= _LANE
    return max(t, _LANE)


def _gram_body(x_ref, w_ref, b_ref, o_ref, g_ref, s_ref, *, nsteps,
               batch_hw, inv_n):
    step = pl.program_id(0) * pl.num_programs(1) + pl.program_id(1)

    @pl.when(step == 0)
    def _():
        g_ref[...] = jnp.zeros_like(g_ref)
        s_ref[...] = jnp.zeros_like(s_ref)

    x = x_ref[0]
    xb = x.astype(jnp.bfloat16)
    g_ref[...] += lax.dot_general(xb, xb, (((1,), (1,)), ((), ())),
                                  preferred_element_type=jnp.float32)
    s_ref[...] += jnp.sum(x, axis=1, keepdims=True)

    @pl.when(step == nsteps - 1)
    def _():
        w = w_ref[...]
        b = b_ref[...]
        wg = jnp.dot(w, g_ref[...], preferred_element_type=jnp.float32)
        quad = jnp.sum(wg * w)
        ws = jnp.dot(w, s_ref[...], preferred_element_type=jnp.float32)
        cross = 2.0 * jnp.sum(ws * b)
        const = batch_hw * jnp.sum(b * b)
        o_ref[...] = jnp.reshape((quad + cross + const) * inv_n, (1, 1))


def kernel(img, weight, bias):
    N, C, H, W = img.shape
    Cout = weight.shape[0]
    hw = H * W
    hw_pad = -(-hw // _LANE) * _LANE
    x3 = img.reshape(N, C, hw)
    if hw_pad != hw:
        x3 = jnp.pad(x3, ((0, 0), (0, 0), (0, hw_pad - hw)))
    T = _pick_tile(hw_pad, 16384)
    steps_j = hw_pad // T

    out = pl.pallas_call(
        functools.partial(_gram_body, nsteps=N * steps_j,
                          batch_hw=float(N * hw), inv_n=1.0 / float(N * Cout * hw)),
        out_shape=jax.ShapeDtypeStruct((1, 1), jnp.float32),
        grid=(N, steps_j),
        in_specs=[
            pl.BlockSpec((1, C, T), lambda n, j: (n, 0, j)),
            pl.BlockSpec((Cout, C), lambda n, j: (0, 0)),
            pl.BlockSpec((Cout, 1), lambda n, j: (0, 0)),
        ],
        out_specs=pl.BlockSpec((1, 1), lambda n, j: (0, 0)),
        scratch_shapes=[
            pltpu.VMEM((C, C), jnp.float32),
            pltpu.VMEM((C, 1), jnp.float32),
        ],
        compiler_params=pltpu.CompilerParams(
            dimension_semantics=("arbitrary", "arbitrary"),
            vmem_limit_bytes=48 * 1024 * 1024),
    )(x3, weight, bias)
    return out[0, 0]
